# baseline (speedup 1.0000x reference)
import functools

import jax
import jax.numpy as jnp
from jax import lax
from jax.experimental import pallas as pl
from jax.experimental.pallas import tpu as pltpu
from jax.experimental.pallas import tpu_sc as plsc

_NC, _NS = 2, 16
_NW = _NC * _NS
_L = 128


@functools.lru_cache(maxsize=None)
def _build(batch: int, hist: int, vocab: int, dim: int):
    assert batch == _NW * _L and hist % 8 == 0 and dim == 32
    n_tr = hist // 8
    n_bc = batch // _L
    tbl_rows = vocab * dim // _L

    mesh = plsc.VectorSubcoreMesh(core_axis_name="c", subcore_axis_name="s")

    @functools.partial(
        pl.kernel,
        out_type=jax.ShapeDtypeStruct((hist, 4, n_bc, 8, _L), jnp.float32),
        mesh=mesh,
        scratch_types=[
            pltpu.VMEM((n_tr, 8, _L), jnp.int32),
            [pltpu.VMEM((_L,), jnp.int32) for _ in range(2)],
            [pltpu.VMEM((_L, _L), jnp.float32) for _ in range(2)],
            [pltpu.VMEM((4, 8, _L), jnp.float32) for _ in range(2)],
            [pltpu.SemaphoreType.DMA for _ in range(2)],
            [pltpu.SemaphoreType.DMA for _ in range(2)],
        ],
        compiler_params=pltpu.CompilerParams(
            use_tc_tiling_on_sc=True, needs_layout_passes=False),
    )
    def gather_kernel(idx4, tbl, out5, idx_v, i2s, gbufs, obs, gsems, wsems):
        wid = lax.axis_index("s") * _NC + lax.axis_index("c")
        pltpu.sync_copy(idx4.at[:, wid], idx_v)

        lvecs = [lax.iota(jnp.int32, 16) + 16 * g for g in range(8)]

        def fire_gather(tr, s, b):
            for g in range(8):
                iv = idx_v[tr, s, pl.ds(16 * g, 16)]
                i2s[b][pl.ds(16 * g, 16)] = lax.shift_right_logical(iv, 2)
            pltpu.async_copy(tbl.at[i2s[b]], gbufs[b], gsems[b])

        def wait_gather(b):
            pltpu.make_async_copy(tbl.at[i2s[b]], gbufs[b], gsems[b]).wait()

        def select(tr, s, b):
            cvecs = []
            for g in range(8):
                iv = idx_v[tr, s, pl.ds(16 * g, 16)]
                cvecs.append((iv & 3) * 32)

            def dbody(d, carry):
                dr = lax.shift_right_logical(d, 3)
                ds_ = d & 7
                for g in range(8):
                    v = plsc.load_gather(gbufs[b], [lvecs[g], cvecs[g] + d])
                    obs[b][dr, ds_, pl.ds(16 * g, 16)] = v
                return carry

            lax.fori_loop(0, dim, dbody, 0)

        def fire_wb(tr, s, b):
            t = tr * 8 + s
            pltpu.async_copy(obs[b], out5.at[t, pl.ds(0, 4), wid], wsems[b])

        def wait_wb(b):
            pltpu.make_async_copy(
                obs[b], out5.at[0, pl.ds(0, 4), 0], wsems[b]).wait()

        fire_gather(0, 0, 0)
        fire_gather(0, 1, 1)

        for s in range(8):
            b = s & 1
            wait_gather(b)
            if s >= 2:
                wait_wb(b)
            select(0, s, b)
            fire_wb(0, s, b)
            t2 = s + 2
            fire_gather(t2 // 8, t2 % 8, b)

        def trbody(tr, carry):
            for s in range(8):
                b = s & 1
                wait_gather(b)
                wait_wb(b)
                select(tr, s, b)
                fire_wb(tr, s, b)
                t2 = jnp.minimum(tr * 8 + s + 2, hist - 1)
                fire_gather(
                    lax.shift_right_logical(t2, 3).astype(jnp.int32),
                    (t2 & 7).astype(jnp.int32), b)
            return carry

        lax.fori_loop(1, n_tr, trbody, 0)

        wait_gather(0)
        wait_gather(1)
        wait_wb(0)
        wait_wb(1)

    return gather_kernel


def kernel(inputs, weight):
    batch, hist = inputs.shape
    vocab, dim = weight.shape
    idx4 = (inputs.astype(jnp.int32)
            .reshape(batch // _L, _L, hist // 8, 8)
            .transpose(2, 0, 3, 1))
    table = weight.reshape(vocab * dim // _L, _L)
    fn = _build(batch, hist, vocab, dim)
    out5 = fn(idx4, table)
    y3 = out5.transpose(0, 1, 3, 2, 4).reshape(hist, dim, batch)
    return y3.transpose(2, 0, 1)
